# Initial kernel scaffold; baseline (speedup 1.0000x reference)
import jax
import jax.numpy as jnp
from jax.experimental import pallas as pl


def kernel(x, edge_index, W1, b1, W2, b2, Wfc, bfc):
    raise NotImplementedError("write your pallas kernel here")


# speedup vs baseline: 12.1240x; 12.1240x over previous
import functools

import jax
import jax.numpy as jnp
from jax import lax
from jax.experimental import pallas as pl
from jax.experimental.pallas import tpu as pltpu
from jax.experimental.pallas import tpu_sc as plsc

NC = 2
NS = 16
L = 16
NW = NC * NS

N_RAW = 10000
E_RAW = 320000
D = 128

N_PAD = 10240
E_TOT = E_RAW + N_RAW
CH = 128
G_PER_TILE = -(-E_TOT // (NW * CH))
E_PAD = G_PER_TILE * NW * CH
ROWS_PER_TILE = N_PAD // NS
ICH = 1296

def _z16():
    return jnp.zeros((L,), jnp.float32)


def _mesh():
    return plsc.VectorSubcoreMesh(
        core_axis_name="c", subcore_axis_name="s", num_cores=NC, num_subcores=NS
    )


DW = 128


def _deg_body(dst_hbm, deg_hbm, idx_v, val_v, deg_sh):
    c = lax.axis_index("c")
    s = lax.axis_index("s")
    wid = c * NS + s

    @pl.loop(0, CH)
    def _zf(r):
        for k in range(DW // L):
            val_v[r, pl.ds(k * L, L)] = _z16()

    @pl.loop(0, ROWS_PER_TILE // CH)
    def _zs(j):
        pltpu.sync_copy(val_v, deg_sh.at[pl.ds(s * ROWS_PER_TILE + j * CH, CH)])

    @pl.loop(0, CH)
    def _of(r):
        for k in range(DW // L):
            val_v[r, pl.ds(k * L, L)] = _z16() + 1.0

    plsc.subcore_barrier()

    gpt = E_PAD // (NW * CH)

    @pl.loop(0, gpt)
    def _edges(g):
        base = (wid * gpt + g) * CH
        pltpu.sync_copy(dst_hbm.at[pl.ds(base, CH)], idx_v)
        pltpu.sync_copy(val_v, deg_sh.at[idx_v], add=True)

    plsc.subcore_barrier()

    @pl.loop(0, ROWS_PER_TILE // CH)
    def _out(j):
        r0 = s * ROWS_PER_TILE + j * CH
        pltpu.sync_copy(deg_sh.at[pl.ds(r0, CH)], val_v)
        pltpu.sync_copy(val_v, deg_hbm.at[c, pl.ds(r0, CH)])


def _deg(dst):
    f = functools.partial(
        pl.kernel,
        out_type=jax.ShapeDtypeStruct((NC, N_PAD, DW), jnp.float32),
        mesh=_mesh(),
        scratch_types=[
            pltpu.VMEM((CH,), jnp.int32),
            pltpu.VMEM((CH, DW), jnp.float32),
            pltpu.VMEM_SHARED((N_PAD, DW), jnp.float32),
        ],
    )(_deg_body)
    return f(dst)


def _prop_body(hp_hbm, src_hbm, dst_hbm, out_hbm, sidx_v, didx_v, rows_v, sem,
               acc_sh):
    c = lax.axis_index("c")
    s = lax.axis_index("s")
    wid = c * NS + s

    @pl.loop(0, CH)
    def _zr(r):
        for k in range(D // L):
            rows_v[r, pl.ds(k * L, L)] = _z16()

    @pl.loop(0, ROWS_PER_TILE // CH)
    def _zs(j):
        pltpu.sync_copy(rows_v, acc_sh.at[pl.ds(s * ROWS_PER_TILE + j * CH, CH)])

    plsc.subcore_barrier()

    @pl.loop(0, G_PER_TILE)
    def _edges(g):
        base = (wid * G_PER_TILE + g) * CH
        pltpu.sync_copy(src_hbm.at[pl.ds(base, CH)], sidx_v)
        pltpu.sync_copy(dst_hbm.at[pl.ds(base, CH)], didx_v)
        pltpu.async_copy(hp_hbm.at[sidx_v], rows_v, sem).wait()
        pltpu.sync_copy(rows_v, acc_sh.at[didx_v], add=True)

    plsc.subcore_barrier()

    @pl.loop(0, ROWS_PER_TILE // CH)
    def _out(j):
        r0 = s * ROWS_PER_TILE + j * CH
        pltpu.sync_copy(acc_sh.at[pl.ds(r0, CH)], rows_v)
        pltpu.sync_copy(rows_v, out_hbm.at[c, pl.ds(r0, CH)])


def _prop(hp, src, dst):
    f = functools.partial(
        pl.kernel,
        out_type=jax.ShapeDtypeStruct((NC, N_PAD, D), jnp.float32),
        mesh=_mesh(),
        scratch_types=[
            pltpu.VMEM((CH,), jnp.int32),
            pltpu.VMEM((CH,), jnp.int32),
            pltpu.VMEM((CH, D), jnp.float32),
            pltpu.SemaphoreType.DMA,
            pltpu.VMEM_SHARED((N_PAD, D), jnp.float32),
        ],
    )(_prop_body)
    return f(hp, src, dst)


def _dis(d0, d1):
    deg = d0 + d1
    return jnp.where(deg > 0, lax.rsqrt(jnp.maximum(deg, 1e-12)), 0.0)


def _mm_in_body(x_ref, w_ref, d0_ref, d1_ref, o_ref):
    h = jnp.dot(x_ref[...], w_ref[...], preferred_element_type=jnp.float32)
    o_ref[...] = h * _dis(d0_ref[...], d1_ref[...])


def _mm_mid_body(a0_ref, a1_ref, d0_ref, d1_ref, b_ref, w_ref, o_ref):
    dis = _dis(d0_ref[...], d1_ref[...])
    pre = jnp.maximum(dis * (a0_ref[...] + a1_ref[...]) + b_ref[...], 0.0)
    h = jnp.dot(pre, w_ref[...], preferred_element_type=jnp.float32)
    o_ref[...] = h * dis


def _mm_fin_body(a0_ref, a1_ref, d0_ref, d1_ref, b_ref, w_ref, bo_ref, o_ref):
    dis = _dis(d0_ref[...], d1_ref[...])
    pre = jnp.maximum(dis * (a0_ref[...] + a1_ref[...]) + b_ref[...], 0.0)
    h = jnp.dot(pre, w_ref[...], preferred_element_type=jnp.float32)
    o_ref[...] = h + bo_ref[...]


_BM = 2048


def _row_spec():
    return pl.BlockSpec((_BM, D), lambda i: (i, 0))


def _d_spec():
    return pl.BlockSpec((_BM, 1), lambda i: (i, 0))


def _w_spec():
    return pl.BlockSpec((D, D), lambda i: (0, 0))


def _b_spec():
    return pl.BlockSpec((1, D), lambda i: (0, 0))


def _out_sds():
    return jax.ShapeDtypeStruct((N_PAD, D), jnp.float32)


def _mm_in(x, w, d0, d1):
    return pl.pallas_call(
        _mm_in_body,
        grid=(N_PAD // _BM,),
        in_specs=[_row_spec(), _w_spec(), _d_spec(), _d_spec()],
        out_specs=_row_spec(),
        out_shape=_out_sds(),
    )(x, w, d0, d1)


def _mm_mid(a0, a1, d0, d1, b, w):
    return pl.pallas_call(
        _mm_mid_body,
        grid=(N_PAD // _BM,),
        in_specs=[_row_spec(), _row_spec(), _d_spec(), _d_spec(), _b_spec(),
                  _w_spec()],
        out_specs=_row_spec(),
        out_shape=_out_sds(),
    )(a0, a1, d0, d1, b, w)


def _mm_fin(a0, a1, d0, d1, b, w, bo):
    return pl.pallas_call(
        _mm_fin_body,
        grid=(N_PAD // _BM,),
        in_specs=[_row_spec(), _row_spec(), _d_spec(), _d_spec(), _b_spec(),
                  _w_spec(), _b_spec()],
        out_specs=_row_spec(),
        out_shape=_out_sds(),
    )(a0, a1, d0, d1, b, w, bo)


def kernel(x, edge_index, W1, b1, W2, b2, Wfc, bfc):
    n = x.shape[0]
    loop_idx = jnp.arange(n, dtype=jnp.int32)
    pad_e = E_PAD - E_RAW - n
    pad_idx = jnp.full((pad_e,), n, dtype=jnp.int32)
    src = jnp.concatenate([edge_index[0].astype(jnp.int32), loop_idx, pad_idx])
    dst = jnp.concatenate([edge_index[1].astype(jnp.int32), loop_idx, pad_idx])

    x_pad = jnp.zeros((N_PAD, D), jnp.float32).at[:n].set(x)

    deg = _deg(dst)
    d0 = deg[0, :, 0:1]
    d1 = deg[1, :, 0:1]

    b1r = b1[None, :]
    b2r = b2[None, :]
    bfr = bfc[None, :]

    h1 = _mm_in(x_pad, W1, d0, d1)
    a1 = _prop(h1, src, dst)
    h2 = _mm_mid(a1[0], a1[1], d0, d1, b1r, W2)
    a2 = _prop(h2, src, dst)
    y = _mm_fin(a2[0], a2[1], d0, d1, b2r, Wfc, bfr)
    return y[:n]


# speedup vs baseline: 16.7543x; 1.3819x over previous
import functools

import jax
import jax.numpy as jnp
from jax import lax
from jax.experimental import pallas as pl
from jax.experimental.pallas import tpu as pltpu
from jax.experimental.pallas import tpu_sc as plsc

NC = 2
NS = 16
L = 16
NW = NC * NS

N_RAW = 10000
E_RAW = 320000
D = 128

N_PAD = 10240
E_TOT = E_RAW + N_RAW
CH = 128
G_PER_TILE = -(-E_TOT // (NW * CH))
G_PAD = 88
E_PAD = G_PAD * NW * CH
ROWS_PER_TILE = N_PAD // NS
SLAB = 48
SLAB_B = 40
KB = G_PER_TILE - SLAB

def _z16():
    return jnp.zeros((L,), jnp.float32)


def _mesh():
    return plsc.VectorSubcoreMesh(
        core_axis_name="c", subcore_axis_name="s", num_cores=NC, num_subcores=NS
    )


DW = 128


def _deg_body(dst3_hbm, deg_hbm, dslab_v, val_v, sem0, sem1, deg_sh):
    c = lax.axis_index("c")
    s = lax.axis_index("s")
    wid = c * NS + s

    @pl.loop(0, CH)
    def _zf(r):
        for k in range(DW // L):
            val_v[r, pl.ds(k * L, L)] = _z16()

    @pl.loop(0, ROWS_PER_TILE // CH)
    def _zs(j):
        pltpu.sync_copy(val_v, deg_sh.at[pl.ds(s * ROWS_PER_TILE + j * CH, CH)])

    @pl.loop(0, CH)
    def _of(r):
        for k in range(DW // L):
            val_v[r, pl.ds(k * L, L)] = _z16() + 1.0

    plsc.subcore_barrier()

    def scat(g, sem):
        pltpu.async_copy(val_v, deg_sh.at[dslab_v.at[g]], sem, add=True)

    def wait_s(sem):
        pltpu.make_async_copy(val_v, deg_sh.at[dslab_v.at[0]], sem).wait()

    def sweep(k):
        scat(0, sem0)
        scat(1, sem1)

        @pl.loop(1, k // 2)
        def _edges(m):
            wait_s(sem0)
            scat(2 * m, sem0)
            wait_s(sem1)
            scat(2 * m + 1, sem1)

        if k % 2 == 1:
            wait_s(sem0)
            scat(k - 1, sem0)
        wait_s(sem0)
        wait_s(sem1)

    pltpu.sync_copy(dst3_hbm.at[wid, pl.ds(0, SLAB)], dslab_v)
    sweep(SLAB)
    pltpu.sync_copy(dst3_hbm.at[wid, pl.ds(SLAB, SLAB_B)],
                    dslab_v.at[pl.ds(0, SLAB_B)])
    sweep(KB)

    plsc.subcore_barrier()

    @pl.loop(0, ROWS_PER_TILE // CH)
    def _out(j):
        r0 = s * ROWS_PER_TILE + j * CH
        pltpu.sync_copy(deg_sh.at[pl.ds(r0, CH)], val_v)
        pltpu.sync_copy(val_v, deg_hbm.at[c, pl.ds(r0, CH)])


def _deg(dst3):
    f = functools.partial(
        pl.kernel,
        out_type=jax.ShapeDtypeStruct((NC, N_PAD, DW), jnp.float32),
        mesh=_mesh(),
        scratch_types=[
            pltpu.VMEM((SLAB, CH), jnp.int32),
            pltpu.VMEM((CH, DW), jnp.float32),
            pltpu.SemaphoreType.DMA,
            pltpu.SemaphoreType.DMA,
            pltpu.VMEM_SHARED((N_PAD, DW), jnp.float32),
        ],
    )(_deg_body)
    return f(dst3)


def _prop_body(hp_hbm, src3_hbm, dst3_hbm, out_hbm, sslab_v, dslab_v,
               rows0_v, rows1_v, gsem0, gsem1, ssem0, ssem1, acc_sh):
    c = lax.axis_index("c")
    s = lax.axis_index("s")
    wid = c * NS + s

    @pl.loop(0, CH)
    def _zr(r):
        for k in range(D // L):
            rows0_v[r, pl.ds(k * L, L)] = _z16()

    @pl.loop(0, ROWS_PER_TILE // CH)
    def _zs(j):
        pltpu.sync_copy(rows0_v, acc_sh.at[pl.ds(s * ROWS_PER_TILE + j * CH, CH)])

    plsc.subcore_barrier()

    def gath(g, buf, sem):
        pltpu.async_copy(hp_hbm.at[sslab_v.at[g]], buf, sem)

    def wait_g(buf, sem):
        pltpu.make_async_copy(hp_hbm.at[sslab_v.at[0]], buf, sem).wait()

    def scat(g, buf, sem):
        pltpu.async_copy(buf, acc_sh.at[dslab_v.at[g]], sem, add=True)

    def wait_s(buf, sem):
        pltpu.make_async_copy(buf, acc_sh.at[dslab_v.at[0]], sem).wait()

    def sweep(k):
        gath(0, rows0_v, gsem0)
        gath(1, rows1_v, gsem1)
        wait_g(rows0_v, gsem0)
        scat(0, rows0_v, ssem0)
        wait_g(rows1_v, gsem1)
        scat(1, rows1_v, ssem1)

        @pl.loop(1, k // 2)
        def _pairs(m):
            wait_s(rows0_v, ssem0)
            gath(2 * m, rows0_v, gsem0)
            wait_s(rows1_v, ssem1)
            gath(2 * m + 1, rows1_v, gsem1)
            wait_g(rows0_v, gsem0)
            scat(2 * m, rows0_v, ssem0)
            wait_g(rows1_v, gsem1)
            scat(2 * m + 1, rows1_v, ssem1)

        if k % 2 == 1:
            wait_s(rows0_v, ssem0)
            gath(k - 1, rows0_v, gsem0)
            wait_g(rows0_v, gsem0)
            scat(k - 1, rows0_v, ssem0)
        wait_s(rows0_v, ssem0)
        wait_s(rows1_v, ssem1)

    pltpu.sync_copy(src3_hbm.at[wid, pl.ds(0, SLAB)], sslab_v)
    pltpu.sync_copy(dst3_hbm.at[wid, pl.ds(0, SLAB)], dslab_v)
    sweep(SLAB)
    pltpu.sync_copy(src3_hbm.at[wid, pl.ds(SLAB, SLAB_B)],
                    sslab_v.at[pl.ds(0, SLAB_B)])
    pltpu.sync_copy(dst3_hbm.at[wid, pl.ds(SLAB, SLAB_B)],
                    dslab_v.at[pl.ds(0, SLAB_B)])
    sweep(KB)

    plsc.subcore_barrier()

    @pl.loop(0, ROWS_PER_TILE // CH)
    def _out(j):
        r0 = s * ROWS_PER_TILE + j * CH
        pltpu.sync_copy(acc_sh.at[pl.ds(r0, CH)], rows0_v)
        pltpu.sync_copy(rows0_v, out_hbm.at[c, pl.ds(r0, CH)])


def _prop(hp, src3, dst3):
    f = functools.partial(
        pl.kernel,
        out_type=jax.ShapeDtypeStruct((NC, N_PAD, D), jnp.float32),
        mesh=_mesh(),
        scratch_types=[
            pltpu.VMEM((SLAB, CH), jnp.int32),
            pltpu.VMEM((SLAB, CH), jnp.int32),
            pltpu.VMEM((CH, D), jnp.float32),
            pltpu.VMEM((CH, D), jnp.float32),
            pltpu.SemaphoreType.DMA,
            pltpu.SemaphoreType.DMA,
            pltpu.SemaphoreType.DMA,
            pltpu.SemaphoreType.DMA,
            pltpu.VMEM_SHARED((N_PAD, D), jnp.float32),
        ],
    )(_prop_body)
    return f(hp, src3, dst3)


def _dis(d0, d1):
    deg = d0 + d1
    return jnp.where(deg > 0, lax.rsqrt(jnp.maximum(deg, 1e-12)), 0.0)


def _mm_in_body(x_ref, w_ref, d0_ref, d1_ref, o_ref):
    h = jnp.dot(x_ref[...], w_ref[...], preferred_element_type=jnp.float32)
    o_ref[...] = h * _dis(d0_ref[...], d1_ref[...])


def _mm_mid_body(a0_ref, a1_ref, d0_ref, d1_ref, b_ref, w_ref, o_ref):
    dis = _dis(d0_ref[...], d1_ref[...])
    pre = jnp.maximum(dis * (a0_ref[...] + a1_ref[...]) + b_ref[...], 0.0)
    h = jnp.dot(pre, w_ref[...], preferred_element_type=jnp.float32)
    o_ref[...] = h * dis


def _mm_fin_body(a0_ref, a1_ref, d0_ref, d1_ref, b_ref, w_ref, bo_ref, o_ref):
    dis = _dis(d0_ref[...], d1_ref[...])
    pre = jnp.maximum(dis * (a0_ref[...] + a1_ref[...]) + b_ref[...], 0.0)
    h = jnp.dot(pre, w_ref[...], preferred_element_type=jnp.float32)
    o_ref[...] = h + bo_ref[...]


_BM = 2048


def _row_spec():
    return pl.BlockSpec((_BM, D), lambda i: (i, 0))


def _d_spec():
    return pl.BlockSpec((_BM, 1), lambda i: (i, 0))


def _w_spec():
    return pl.BlockSpec((D, D), lambda i: (0, 0))


def _b_spec():
    return pl.BlockSpec((1, D), lambda i: (0, 0))


def _out_sds():
    return jax.ShapeDtypeStruct((N_PAD, D), jnp.float32)


def _mm_in(x, w, d0, d1):
    return pl.pallas_call(
        _mm_in_body,
        grid=(N_PAD // _BM,),
        in_specs=[_row_spec(), _w_spec(), _d_spec(), _d_spec()],
        out_specs=_row_spec(),
        out_shape=_out_sds(),
    )(x, w, d0, d1)


def _mm_mid(a0, a1, d0, d1, b, w):
    return pl.pallas_call(
        _mm_mid_body,
        grid=(N_PAD // _BM,),
        in_specs=[_row_spec(), _row_spec(), _d_spec(), _d_spec(), _b_spec(),
                  _w_spec()],
        out_specs=_row_spec(),
        out_shape=_out_sds(),
    )(a0, a1, d0, d1, b, w)


def _mm_fin(a0, a1, d0, d1, b, w, bo):
    return pl.pallas_call(
        _mm_fin_body,
        grid=(N_PAD // _BM,),
        in_specs=[_row_spec(), _row_spec(), _d_spec(), _d_spec(), _b_spec(),
                  _w_spec(), _b_spec()],
        out_specs=_row_spec(),
        out_shape=_out_sds(),
    )(a0, a1, d0, d1, b, w, bo)


def kernel(x, edge_index, W1, b1, W2, b2, Wfc, bfc):
    n = x.shape[0]
    loop_idx = jnp.arange(n, dtype=jnp.int32)
    pad_e = G_PER_TILE * NW * CH - E_RAW - n
    pad_idx = jnp.full((pad_e,), n, dtype=jnp.int32)
    src = jnp.concatenate([edge_index[0].astype(jnp.int32), loop_idx, pad_idx])
    dst = jnp.concatenate([edge_index[1].astype(jnp.int32), loop_idx, pad_idx])
    src = jnp.pad(src.reshape(NW, G_PER_TILE, CH),
                  ((0, 0), (0, G_PAD - G_PER_TILE), (0, 0)),
                  constant_values=n)
    dst = jnp.pad(dst.reshape(NW, G_PER_TILE, CH),
                  ((0, 0), (0, G_PAD - G_PER_TILE), (0, 0)),
                  constant_values=n)

    x_pad = jnp.zeros((N_PAD, D), jnp.float32).at[:n].set(x)

    deg = _deg(dst)
    d0 = deg[0, :, 0:1]
    d1 = deg[1, :, 0:1]

    b1r = b1[None, :]
    b2r = b2[None, :]
    bfr = bfc[None, :]

    h1 = _mm_in(x_pad, W1, d0, d1)
    a1 = _prop(h1, src, dst)
    h2 = _mm_mid(a1[0], a1[1], d0, d1, b1r, W2)
    a2 = _prop(h2, src, dst)
    y = _mm_fin(a2[0], a2[1], d0, d1, b2r, Wfc, bfr)
    return y[:n]


# speedup vs baseline: 17.7635x; 1.0602x over previous
import functools

import jax
import jax.numpy as jnp
from jax import lax
from jax.experimental import pallas as pl
from jax.experimental.pallas import tpu as pltpu
from jax.experimental.pallas import tpu_sc as plsc

NC = 2
NS = 16
L = 16
NW = NC * NS

N_RAW = 10000
E_RAW = 320000
D = 128

N_PAD = 10240
E_TOT = E_RAW + N_RAW
CH = 64
G_PER_TILE = -(-E_TOT // (NW * CH))
G_PAD = 176
E_PAD = G_PAD * NW * CH
ROWS_PER_TILE = N_PAD // NS
SLAB = 48
PHASES = ((0, 48, 48), (48, 48, 48), (96, 48, 48), (144, 32, 18))

def _z16():
    return jnp.zeros((L,), jnp.float32)


def _mesh():
    return plsc.VectorSubcoreMesh(
        core_axis_name="c", subcore_axis_name="s", num_cores=NC, num_subcores=NS
    )


DW = 128


def _deg_body(dst3_hbm, deg_hbm, dslab_v, val_v, sem0, sem1, deg_sh):
    c = lax.axis_index("c")
    s = lax.axis_index("s")
    wid = c * NS + s

    @pl.loop(0, CH)
    def _zf(r):
        for k in range(DW // L):
            val_v[r, pl.ds(k * L, L)] = _z16()

    @pl.loop(0, ROWS_PER_TILE // CH)
    def _zs(j):
        pltpu.sync_copy(val_v, deg_sh.at[pl.ds(s * ROWS_PER_TILE + j * CH, CH)])

    @pl.loop(0, CH)
    def _of(r):
        for k in range(DW // L):
            val_v[r, pl.ds(k * L, L)] = _z16() + 1.0

    plsc.subcore_barrier()

    def scat(g, sem):
        pltpu.async_copy(val_v, deg_sh.at[dslab_v.at[g]], sem, add=True)

    def wait_s(sem):
        pltpu.make_async_copy(val_v, deg_sh.at[dslab_v.at[0]], sem).wait()

    def sweep(k):
        scat(0, sem0)
        scat(1, sem1)

        @pl.loop(1, k // 2)
        def _edges(m):
            wait_s(sem0)
            scat(2 * m, sem0)
            wait_s(sem1)
            scat(2 * m + 1, sem1)

        if k % 2 == 1:
            wait_s(sem0)
            scat(k - 1, sem0)
        wait_s(sem0)
        wait_s(sem1)

    for off, rows, k in PHASES:
        pltpu.sync_copy(dst3_hbm.at[wid, pl.ds(off, rows)],
                        dslab_v.at[pl.ds(0, rows)])
        sweep(k)

    plsc.subcore_barrier()

    @pl.loop(0, ROWS_PER_TILE // CH)
    def _out(j):
        r0 = s * ROWS_PER_TILE + j * CH
        pltpu.sync_copy(deg_sh.at[pl.ds(r0, CH)], val_v)
        pltpu.sync_copy(val_v, deg_hbm.at[c, pl.ds(r0, CH)])


def _deg(dst3):
    f = functools.partial(
        pl.kernel,
        out_type=jax.ShapeDtypeStruct((NC, N_PAD, DW), jnp.float32),
        mesh=_mesh(),
        scratch_types=[
            pltpu.VMEM((SLAB, CH), jnp.int32),
            pltpu.VMEM((CH, DW), jnp.float32),
            pltpu.SemaphoreType.DMA,
            pltpu.SemaphoreType.DMA,
            pltpu.VMEM_SHARED((N_PAD, DW), jnp.float32),
        ],
    )(_deg_body)
    return f(dst3)


def _prop_body(hp_hbm, src3_hbm, dst3_hbm, out_hbm, sslab_v, dslab_v,
               rows0_v, rows1_v, rows2_v, rows3_v,
               gsem0, gsem1, gsem2, gsem3, ssem0, ssem1, ssem2, ssem3, acc_sh):
    c = lax.axis_index("c")
    s = lax.axis_index("s")
    wid = c * NS + s

    @pl.loop(0, CH)
    def _zr(r):
        for k in range(D // L):
            rows0_v[r, pl.ds(k * L, L)] = _z16()

    @pl.loop(0, ROWS_PER_TILE // CH)
    def _zs(j):
        pltpu.sync_copy(rows0_v, acc_sh.at[pl.ds(s * ROWS_PER_TILE + j * CH, CH)])

    plsc.subcore_barrier()

    def gath(g, buf, sem):
        pltpu.async_copy(hp_hbm.at[sslab_v.at[g]], buf, sem)

    def wait_g(buf, sem):
        pltpu.make_async_copy(hp_hbm.at[sslab_v.at[0]], buf, sem).wait()

    def scat(g, buf, sem):
        pltpu.async_copy(buf, acc_sh.at[dslab_v.at[g]], sem, add=True)

    def wait_s(buf, sem):
        pltpu.make_async_copy(buf, acc_sh.at[dslab_v.at[0]], sem).wait()

    bufs = [rows0_v, rows1_v, rows2_v, rows3_v]
    gsems = [gsem0, gsem1, gsem2, gsem3]
    ssems = [ssem0, ssem1, ssem2, ssem3]

    def sweep(k):
        for i in range(4):
            gath(i, bufs[i], gsems[i])
        for i in range(4):
            wait_g(bufs[i], gsems[i])
            scat(i, bufs[i], ssems[i])

        @pl.loop(1, k // 4)
        def _quads(m):
            for i in range(4):
                wait_s(bufs[i], ssems[i])
                gath(4 * m + i, bufs[i], gsems[i])
            for i in range(4):
                wait_g(bufs[i], gsems[i])
                scat(4 * m + i, bufs[i], ssems[i])

        for i in range(k % 4):
            wait_s(bufs[i], ssems[i])
            gath((k // 4) * 4 + i, bufs[i], gsems[i])
            wait_g(bufs[i], gsems[i])
            scat((k // 4) * 4 + i, bufs[i], ssems[i])
        for i in range(4):
            wait_s(bufs[i], ssems[i])

    for off, rows, k in PHASES:
        pltpu.sync_copy(src3_hbm.at[wid, pl.ds(off, rows)],
                        sslab_v.at[pl.ds(0, rows)])
        pltpu.sync_copy(dst3_hbm.at[wid, pl.ds(off, rows)],
                        dslab_v.at[pl.ds(0, rows)])
        sweep(k)

    plsc.subcore_barrier()

    @pl.loop(0, ROWS_PER_TILE // CH)
    def _out(j):
        r0 = s * ROWS_PER_TILE + j * CH
        pltpu.sync_copy(acc_sh.at[pl.ds(r0, CH)], rows0_v)
        pltpu.sync_copy(rows0_v, out_hbm.at[c, pl.ds(r0, CH)])


def _prop(hp, src3, dst3):
    f = functools.partial(
        pl.kernel,
        out_type=jax.ShapeDtypeStruct((NC, N_PAD, D), jnp.float32),
        mesh=_mesh(),
        scratch_types=[
            pltpu.VMEM((SLAB, CH), jnp.int32),
            pltpu.VMEM((SLAB, CH), jnp.int32),
            pltpu.VMEM((CH, D), jnp.float32),
            pltpu.VMEM((CH, D), jnp.float32),
            pltpu.VMEM((CH, D), jnp.float32),
            pltpu.VMEM((CH, D), jnp.float32),
            pltpu.SemaphoreType.DMA,
            pltpu.SemaphoreType.DMA,
            pltpu.SemaphoreType.DMA,
            pltpu.SemaphoreType.DMA,
            pltpu.SemaphoreType.DMA,
            pltpu.SemaphoreType.DMA,
            pltpu.SemaphoreType.DMA,
            pltpu.SemaphoreType.DMA,
            pltpu.VMEM_SHARED((N_PAD, D), jnp.float32),
        ],
    )(_prop_body)
    return f(hp, src3, dst3)


def _dis(d0, d1):
    deg = d0 + d1
    return jnp.where(deg > 0, lax.rsqrt(jnp.maximum(deg, 1e-12)), 0.0)


def _mm_in_body(x_ref, w_ref, d0_ref, d1_ref, o_ref):
    h = jnp.dot(x_ref[...], w_ref[...], preferred_element_type=jnp.float32)
    o_ref[...] = h * _dis(d0_ref[...], d1_ref[...])


def _mm_mid_body(a0_ref, a1_ref, d0_ref, d1_ref, b_ref, w_ref, o_ref):
    dis = _dis(d0_ref[...], d1_ref[...])
    pre = jnp.maximum(dis * (a0_ref[...] + a1_ref[...]) + b_ref[...], 0.0)
    h = jnp.dot(pre, w_ref[...], preferred_element_type=jnp.float32)
    o_ref[...] = h * dis


def _mm_fin_body(a0_ref, a1_ref, d0_ref, d1_ref, b_ref, w_ref, bo_ref, o_ref):
    dis = _dis(d0_ref[...], d1_ref[...])
    pre = jnp.maximum(dis * (a0_ref[...] + a1_ref[...]) + b_ref[...], 0.0)
    h = jnp.dot(pre, w_ref[...], preferred_element_type=jnp.float32)
    o_ref[...] = h + bo_ref[...]


_BM = 2048


def _row_spec():
    return pl.BlockSpec((_BM, D), lambda i: (i, 0))


def _d_spec():
    return pl.BlockSpec((_BM, 1), lambda i: (i, 0))


def _w_spec():
    return pl.BlockSpec((D, D), lambda i: (0, 0))


def _b_spec():
    return pl.BlockSpec((1, D), lambda i: (0, 0))


def _out_sds():
    return jax.ShapeDtypeStruct((N_PAD, D), jnp.float32)


def _mm_in(x, w, d0, d1):
    return pl.pallas_call(
        _mm_in_body,
        grid=(N_PAD // _BM,),
        in_specs=[_row_spec(), _w_spec(), _d_spec(), _d_spec()],
        out_specs=_row_spec(),
        out_shape=_out_sds(),
    )(x, w, d0, d1)


def _mm_mid(a0, a1, d0, d1, b, w):
    return pl.pallas_call(
        _mm_mid_body,
        grid=(N_PAD // _BM,),
        in_specs=[_row_spec(), _row_spec(), _d_spec(), _d_spec(), _b_spec(),
                  _w_spec()],
        out_specs=_row_spec(),
        out_shape=_out_sds(),
    )(a0, a1, d0, d1, b, w)


def _mm_fin(a0, a1, d0, d1, b, w, bo):
    return pl.pallas_call(
        _mm_fin_body,
        grid=(N_PAD // _BM,),
        in_specs=[_row_spec(), _row_spec(), _d_spec(), _d_spec(), _b_spec(),
                  _w_spec(), _b_spec()],
        out_specs=_row_spec(),
        out_shape=_out_sds(),
    )(a0, a1, d0, d1, b, w, bo)


def kernel(x, edge_index, W1, b1, W2, b2, Wfc, bfc):
    n = x.shape[0]
    loop_idx = jnp.arange(n, dtype=jnp.int32)
    pad_e = G_PER_TILE * NW * CH - E_RAW - n
    pad_idx = jnp.full((pad_e,), n, dtype=jnp.int32)
    src = jnp.concatenate([edge_index[0].astype(jnp.int32), loop_idx, pad_idx])
    dst = jnp.concatenate([edge_index[1].astype(jnp.int32), loop_idx, pad_idx])
    src = jnp.pad(src.reshape(NW, G_PER_TILE, CH),
                  ((0, 0), (0, G_PAD - G_PER_TILE), (0, 0)),
                  constant_values=n)
    dst = jnp.pad(dst.reshape(NW, G_PER_TILE, CH),
                  ((0, 0), (0, G_PAD - G_PER_TILE), (0, 0)),
                  constant_values=n)

    x_pad = jnp.zeros((N_PAD, D), jnp.float32).at[:n].set(x)

    deg = _deg(dst)
    d0 = deg[0, :, 0:1]
    d1 = deg[1, :, 0:1]

    b1r = b1[None, :]
    b2r = b2[None, :]
    bfr = bfc[None, :]

    h1 = _mm_in(x_pad, W1, d0, d1)
    a1 = _prop(h1, src, dst)
    h2 = _mm_mid(a1[0], a1[1], d0, d1, b1r, W2)
    a2 = _prop(h2, src, dst)
    y = _mm_fin(a2[0], a2[1], d0, d1, b2r, Wfc, bfr)
    return y[:n]
